# baseline (speedup 1.0000x reference)
import jax
import jax.numpy as jnp
import numpy as np
from jax.experimental import pallas as pl

OBS = 8
PRED = 12
EMB = 64
HID = 128
GS = 8
G = GS * GS
B = 8
L = 64
N = B * L
GSN = 2.0 / (GS - 1)


def _fwd(otr_ref, ot_ref, w_in_ref, b_in_ref, wpr_ref, b_pool_ref,
         wx_ref, wh_ref, ws_ref, b_ifog_ref, w_out_ref, b_out_ref,
         out_ref):
    w0 = w_in_ref[0:1, :]
    w1 = w_in_ref[1:2, :]
    b_in = b_in_ref[0:1, :]
    wpr = wpr_ref[...]
    b_pool = b_pool_ref[0:1, :]
    wx = wx_ref[...]
    wh = wh_ref[...]
    ws = ws_ref[...]
    b_ifog = b_ifog_ref[0:1, :]
    w_out = w_out_ref[...]
    b_out = b_out_ref[0:1, :]

    iota_i = jax.lax.broadcasted_iota(jnp.int32, (L, L), 0)
    iota_j = jax.lax.broadcasted_iota(jnp.int32, (L, L), 1)
    not_eye = (iota_i != iota_j)
    iota_g3 = jax.lax.broadcasted_iota(jnp.int32, (1, G, 1), 1)

    def social_fn(h, pos):
        p = pos.reshape(B, L, 2)
        px = p[:, :, 0]
        py = p[:, :, 1]
        relx = px[:, :, None] - px[:, None, :]
        rely = py[:, :, None] - py[:, None, :]
        gx = (jnp.clip(relx / GSN, -(GS // 2), GS // 2) + GS // 2).astype(jnp.int32)
        gy = (jnp.clip(rely / GSN, -(GS // 2), GS // 2) + GS // 2).astype(jnp.int32)
        valid = (gx < GS) & (gy < GS) & not_eye[None, :, :]
        cellT = jnp.where(valid, gy * GS + gx, -1)

        outs = []
        for b in range(B):
            A = (cellT[b][:, None, :] == iota_g3).astype(jnp.float32)
            A = A.reshape(L * G, L)
            ub = jnp.dot(h[b * L:(b + 1) * L], wpr,
                         preferred_element_type=jnp.float32)
            Bm = jnp.concatenate(
                [ub[:, g * EMB:(g + 1) * EMB][:, None, :] for g in range(G)],
                axis=1)
            Bm = Bm.reshape(L * G, EMB)
            outs.append(jax.lax.dot_general(
                A, Bm, (((0,), (0,)), ((), ())),
                preferred_element_type=jnp.float32))
        sr = jnp.concatenate(outs, axis=0)
        return jnp.maximum(sr + b_pool, 0.0)

    def cell_fn(emb, h, c, social):
        z = (jnp.dot(emb, wx, preferred_element_type=jnp.float32)
             + jnp.dot(h, wh, preferred_element_type=jnp.float32)
             + jnp.dot(social, ws, preferred_element_type=jnp.float32)
             + b_ifog)
        i = jax.nn.sigmoid(z[:, 0 * HID:1 * HID])
        f = jax.nn.sigmoid(z[:, 1 * HID:2 * HID])
        o = jax.nn.sigmoid(z[:, 2 * HID:3 * HID])
        g = jnp.tanh(z[:, 3 * HID:4 * HID])
        c2 = f * c + i * g
        h2 = o * jnp.tanh(c2)
        return h2, c2

    def obs_body(t, hc):
        h, c = hc
        vel = otr_ref[pl.ds(t, 1)][0]
        emb = vel[:, 0:1] * w0 + vel[:, 1:2] * w1 + b_in
        pos = ot_ref[pl.ds(t, 1)][0]
        social = social_fn(h, pos)
        return cell_fn(emb, h, c, social)

    h0 = jnp.zeros((N, HID), jnp.float32)
    c0 = jnp.zeros((N, HID), jnp.float32)
    h, c = jax.lax.fori_loop(0, OBS, obs_body, (h0, c0))
    pos = ot_ref[OBS - 1]

    def pred_body(t, carry):
        h, c, pos = carry
        pv = jnp.dot(h, w_out, preferred_element_type=jnp.float32) + b_out
        out_ref[pl.ds(t, 1)] = pv[None]
        pos2 = pos + pv
        emb = pv[:, 0:1] * w0 + pv[:, 1:2] * w1 + b_in
        social = social_fn(h, pos2)
        h2, c2 = cell_fn(emb, h, c, social)
        return (h2, c2, pos2)

    jax.lax.fori_loop(0, PRED, pred_body, (h, c, pos))


@jax.jit
def kernel(obs_traj_rel, seq_start_end, obs_traj, W_in, b_in, W_pool, b_pool,
           W_i, b_i, W_f, b_f, W_o, b_o, W_g, b_g, W_out, b_out):
    del seq_start_end
    wpr = W_pool.reshape(G, HID, EMB).transpose(1, 0, 2).reshape(HID, G * EMB)
    wall = jnp.concatenate([W_i, W_f, W_o, W_g], axis=1)
    wx = wall[:EMB]
    wh = wall[EMB:EMB + HID]
    ws = wall[EMB + HID:]
    b_ifog = jnp.concatenate([b_i, b_f, b_o, b_g]).reshape(1, 4 * HID)
    return pl.pallas_call(
        _fwd,
        out_shape=jax.ShapeDtypeStruct((PRED, N, 2), jnp.float32),
    )(obs_traj_rel, obs_traj, W_in, b_in.reshape(1, EMB), wpr,
      b_pool.reshape(1, EMB), wx, wh, ws, b_ifog, W_out, b_out.reshape(1, 2))
